# baseline (speedup 1.0000x reference)
import functools

import jax
import jax.numpy as jnp
from jax import lax
from jax.experimental import pallas as pl
from jax.experimental.pallas import tpu as pltpu
from jax.experimental.pallas import tpu_sc as plsc

VOCAB = 100000
EMBED = 128
MAX_LEN = 200
B = 4096
L = 200

NC = 2
NS = 16
NW = NC * NS

GCH = 100
GPR = L // GCH
N_CHUNKS = (B * L) // GCH
RPW = B // NW
CPW = N_CHUNKS // NW
LANES = 16
VPR = EMBED // LANES


def _tec_body(ids_hbm, tok_hbm, pos_hbm, out_hbm,
              idx_all, pos_v, buf_a, buf_b, gsem_a, gsem_b, ssem_a, ssem_b):
    c = lax.axis_index("c")
    s = lax.axis_index("s")
    w = s * NC + c

    pltpu.sync_copy(ids_hbm.at[pl.ds(w * CPW, CPW)], idx_all)
    pltpu.sync_copy(pos_hbm, pos_v)

    def fire_gather(r, buf, sem):
        for g in range(GPR):
            pltpu.async_copy(
                tok_hbm.at[idx_all.at[r * GPR + g]],
                buf.at[pl.ds(g * GCH, GCH)],
                sem,
            )

    def wait_gather(buf, sem):
        pltpu.make_async_copy(tok_hbm.at[pl.ds(0, L)], buf, sem).wait()

    def add_pos(buf):
        def row_body(i, carry):
            r0 = i * 8
            for rr in range(8):
                for cc in range(VPR):
                    sl = pl.ds(cc * LANES, LANES)
                    plsc.addupdate(buf.at[r0 + rr, sl], pos_v[r0 + rr, sl])
            return carry
        lax.fori_loop(0, L // 8, row_body, 0)

    def fire_store(r, buf, sem):
        pltpu.async_copy(buf, out_hbm.at[pl.ds((w * RPW + r) * L, L)], sem)

    def wait_store(buf, sem):
        pltpu.make_async_copy(buf, out_hbm.at[pl.ds(0, L)], sem).wait()

    fire_gather(0, buf_a, gsem_a)

    def body(k, carry):
        r0 = 2 * k
        r1 = r0 + 1

        @pl.when(k > 0)
        def _():
            wait_store(buf_b, ssem_b)

        fire_gather(r1, buf_b, gsem_b)
        wait_gather(buf_a, gsem_a)
        add_pos(buf_a)
        fire_store(r0, buf_a, ssem_a)

        wait_store(buf_a, ssem_a)

        @pl.when(k < RPW // 2 - 1)
        def _():
            fire_gather(r0 + 2, buf_a, gsem_a)

        wait_gather(buf_b, gsem_b)
        add_pos(buf_b)
        fire_store(r1, buf_b, ssem_b)
        return carry

    lax.fori_loop(0, RPW // 2, body, 0)
    wait_store(buf_b, ssem_b)


@jax.jit
def _run(ids2d, token_table, pos_table):
    mesh = plsc.VectorSubcoreMesh(core_axis_name="c", subcore_axis_name="s")
    kern = functools.partial(
        pl.kernel,
        mesh=mesh,
        out_type=jax.ShapeDtypeStruct((B * L, EMBED), jnp.float32),
        scratch_types=[
            pltpu.VMEM((CPW, GCH), jnp.int32),
            pltpu.VMEM((MAX_LEN, EMBED), jnp.float32),
            pltpu.VMEM((L, EMBED), jnp.float32),
            pltpu.VMEM((L, EMBED), jnp.float32),
            pltpu.SemaphoreType.DMA,
            pltpu.SemaphoreType.DMA,
            pltpu.SemaphoreType.DMA,
            pltpu.SemaphoreType.DMA,
        ],
    )(_tec_body)
    return kern(ids2d, token_table, pos_table)


def kernel(input_ids, token_table, pos_table):
    ids2d = input_ids.astype(jnp.int32).reshape(N_CHUNKS, GCH)
    out = _run(ids2d, token_table, pos_table)
    return out.reshape(B, L, EMBED)


# speedup vs baseline: 1.2064x; 1.2064x over previous
import functools

import jax
import jax.numpy as jnp
from jax import lax
from jax.experimental import pallas as pl
from jax.experimental.pallas import tpu as pltpu
from jax.experimental.pallas import tpu_sc as plsc

VOCAB = 100000
EMBED = 128
MAX_LEN = 200
B = 4096
L = 200

NC = 2
NS = 16
NW = NC * NS

GCH = 100
GPR = L // GCH
N_CHUNKS = (B * L) // GCH
RPW = B // NW
CPW = N_CHUNKS // NW
LANES = 16
VPR = EMBED // LANES

NT = RPW // 3


def _tec_body(ids_hbm, tok_hbm, pos_hbm, out_hbm,
              pos_v, buf_a, buf_b, buf_c, idx_a, idx_b, idx_c,
              gsem_a, gsem_b, gsem_c, ssem_a, ssem_b, ssem_c,
              isem_a, isem_b, isem_c):
    c = lax.axis_index("c")
    s = lax.axis_index("s")
    w = s * NC + c

    pltpu.sync_copy(pos_hbm, pos_v)

    def fire_idx(r, idxb, sem):
        pltpu.async_copy(
            ids_hbm.at[pl.ds(w * CPW + r * GPR, GPR)], idxb, sem)

    def wait_idx(idxb, sem):
        pltpu.make_async_copy(ids_hbm.at[pl.ds(0, GPR)], idxb, sem).wait()

    def fire_gather(idxb, buf, sem):
        for g in range(GPR):
            pltpu.async_copy(
                tok_hbm.at[idxb.at[g]],
                buf.at[pl.ds(g * GCH, GCH)],
                sem,
            )

    def wait_gather(buf, sem):
        pltpu.make_async_copy(tok_hbm.at[pl.ds(0, L)], buf, sem).wait()

    def add_pos(buf):
        def row_body(i, carry):
            r0 = i * 8
            for rr in range(8):
                for cc in range(VPR):
                    sl = pl.ds(cc * LANES, LANES)
                    plsc.addupdate(buf.at[r0 + rr, sl], pos_v[r0 + rr, sl])
            return carry
        lax.fori_loop(0, L // 8, row_body, 0)

    def fire_store(r, buf, sem):
        pltpu.async_copy(buf, out_hbm.at[pl.ds((w * RPW + r) * L, L)], sem)

    def wait_store(buf, sem):
        pltpu.make_async_copy(buf, out_hbm.at[pl.ds(0, L)], sem).wait()

    pltpu.sync_copy(ids_hbm.at[pl.ds(w * CPW, GPR)], idx_a)
    pltpu.sync_copy(ids_hbm.at[pl.ds(w * CPW + GPR, GPR)], idx_b)
    fire_gather(idx_a, buf_a, gsem_a)
    fire_gather(idx_b, buf_b, gsem_b)
    fire_idx(2, idx_c, isem_c)

    def body(k, carry):
        r0 = 3 * k

        wait_gather(buf_a, gsem_a)
        fire_idx(r0 + 3, idx_a, isem_a)
        add_pos(buf_a)
        fire_store(r0, buf_a, ssem_a)

        @pl.when(k > 0)
        def _():
            wait_store(buf_c, ssem_c)
        wait_idx(idx_c, isem_c)
        fire_gather(idx_c, buf_c, gsem_c)

        wait_gather(buf_b, gsem_b)
        fire_idx(r0 + 4, idx_b, isem_b)
        add_pos(buf_b)
        fire_store(r0 + 1, buf_b, ssem_b)

        wait_store(buf_a, ssem_a)
        wait_idx(idx_a, isem_a)
        fire_gather(idx_a, buf_a, gsem_a)

        wait_gather(buf_c, gsem_c)

        @pl.when(k < NT - 1)
        def _():
            fire_idx(r0 + 5, idx_c, isem_c)
        add_pos(buf_c)
        fire_store(r0 + 2, buf_c, ssem_c)

        wait_store(buf_b, ssem_b)
        wait_idx(idx_b, isem_b)
        fire_gather(idx_b, buf_b, gsem_b)
        return carry

    lax.fori_loop(0, NT, body, 0)

    wait_gather(buf_a, gsem_a)
    add_pos(buf_a)
    fire_store(RPW - 2, buf_a, ssem_a)

    wait_gather(buf_b, gsem_b)
    add_pos(buf_b)
    fire_store(RPW - 1, buf_b, ssem_b)

    wait_store(buf_c, ssem_c)
    wait_store(buf_a, ssem_a)
    wait_store(buf_b, ssem_b)


@jax.jit
def _run(ids2d, token_table, pos_table):
    mesh = plsc.VectorSubcoreMesh(core_axis_name="c", subcore_axis_name="s")
    kern = functools.partial(
        pl.kernel,
        mesh=mesh,
        out_type=jax.ShapeDtypeStruct((B * L, EMBED), jnp.float32),
        scratch_types=[
            pltpu.VMEM((MAX_LEN, EMBED), jnp.float32),
            pltpu.VMEM((L, EMBED), jnp.float32),
            pltpu.VMEM((L, EMBED), jnp.float32),
            pltpu.VMEM((L, EMBED), jnp.float32),
            pltpu.VMEM((GPR, GCH), jnp.int32),
            pltpu.VMEM((GPR, GCH), jnp.int32),
            pltpu.VMEM((GPR, GCH), jnp.int32),
            pltpu.SemaphoreType.DMA,
            pltpu.SemaphoreType.DMA,
            pltpu.SemaphoreType.DMA,
            pltpu.SemaphoreType.DMA,
            pltpu.SemaphoreType.DMA,
            pltpu.SemaphoreType.DMA,
            pltpu.SemaphoreType.DMA,
            pltpu.SemaphoreType.DMA,
            pltpu.SemaphoreType.DMA,
        ],
    )(_tec_body)
    return kern(ids2d, token_table, pos_table)


def kernel(input_ids, token_table, pos_table):
    ids2d = input_ids.astype(jnp.int32).reshape(N_CHUNKS, GCH)
    out = _run(ids2d, token_table, pos_table)
    return out.reshape(B, L, EMBED)
